# baseline (speedup 1.0000x reference)
import functools

import jax
import jax.numpy as jnp
from jax import lax
from jax.experimental import pallas as pl
from jax.experimental.pallas import tpu as pltpu
from jax.experimental.pallas import tpu_sc as plsc

N = 16384
D = 64
KPAD = 128
NC = 2
NS = 16
NW = NC * NS
CHUNK = N // NW
NB = CHUNK // 128
CW = 16


def _sc_body(emb_hbm, lab_hbm, out_sums, out_sq, out_cnt,
             lab_v, emb_v, sq_v, ones_v, zb64, zb16,
             sh_sums, sh_sq, sh_cnt):
  c = lax.axis_index("c")
  s = lax.axis_index("s")
  wid = s * NC + c
  base = wid * CHUNK

  zv = jnp.zeros((16,), jnp.float32)
  ov = jnp.ones((16,), jnp.float32)

  rows = KPAD // NS
  for i in range(rows):
    for j in range(D // 16):
      zb64[i, pl.ds(j * 16, 16)] = zv
    zb16[i, :] = zv
  pltpu.sync_copy(zb64, sh_sums.at[pl.ds(s * rows, rows)])
  pltpu.sync_copy(zb64, sh_sq.at[pl.ds(s * rows, rows)])
  pltpu.sync_copy(zb16, sh_cnt.at[pl.ds(s * rows, rows)])

  def ones_body(i, carry):
    ones_v[i, :] = ov
    return carry
  lax.fori_loop(0, 128, ones_body, 0)

  pltpu.sync_copy(emb_hbm.at[pl.ds(base, CHUNK)], emb_v)
  for b in range(NB):
    pltpu.sync_copy(lab_hbm.at[pl.ds(base + b * 128, 128)], lab_v.at[b])

  plsc.subcore_barrier()

  for b in range(NB):
    @plsc.parallel_loop(0, 128, unroll=4)
    def row_body(r):
      for j in range(D // 16):
        v = emb_v[b * 128 + r, pl.ds(j * 16, 16)]
        sq_v[r, pl.ds(j * 16, 16)] = v * v

    pltpu.sync_copy(emb_v.at[pl.ds(b * 128, 128)],
                    sh_sums.at[lab_v.at[b]], add=True)
    pltpu.sync_copy(sq_v, sh_sq.at[lab_v.at[b]], add=True)
    pltpu.sync_copy(ones_v, sh_cnt.at[lab_v.at[b]], add=True)

  plsc.subcore_barrier()

  @pl.when(s == 0)
  def _dump():
    pltpu.sync_copy(sh_sums, out_sums.at[c])
    pltpu.sync_copy(sh_sq, out_sq.at[c])
    pltpu.sync_copy(sh_cnt, out_cnt.at[c])


@functools.partial(
    pl.kernel,
    out_type=(
        jax.ShapeDtypeStruct((NC, KPAD, D), jnp.float32),
        jax.ShapeDtypeStruct((NC, KPAD, D), jnp.float32),
        jax.ShapeDtypeStruct((NC, KPAD, CW), jnp.float32),
    ),
    mesh=plsc.VectorSubcoreMesh(
        core_axis_name="c", subcore_axis_name="s",
        num_cores=NC, num_subcores=NS),
    compiler_params=pltpu.CompilerParams(use_tc_tiling_on_sc=False),
    scratch_types=[
        pltpu.VMEM((NB, 128), jnp.int32),
        pltpu.VMEM((CHUNK, D), jnp.float32),
        pltpu.VMEM((128, D), jnp.float32),
        pltpu.VMEM((128, CW), jnp.float32),
        pltpu.VMEM((KPAD // NS, D), jnp.float32),
        pltpu.VMEM((KPAD // NS, CW), jnp.float32),
        pltpu.VMEM_SHARED((KPAD, D), jnp.float32),
        pltpu.VMEM_SHARED((KPAD, D), jnp.float32),
        pltpu.VMEM_SHARED((KPAD, CW), jnp.float32),
    ],
)
def _sc_partials(emb_hbm, lab_hbm, out_sums, out_sq, out_cnt, *scratch):
  _sc_body(emb_hbm, lab_hbm, out_sums, out_sq, out_cnt, *scratch)


def _tc_combine_body(sums_ref, sq_ref, cnt_ref, out_ref):
  sums = sums_ref[0] + sums_ref[1]
  sq = sq_ref[0] + sq_ref[1]
  cnt = cnt_ref[0, :, 0:1] + cnt_ref[1, :, 0:1]
  sumsq = jnp.sum(sq, axis=1, keepdims=True)
  normsq = jnp.sum(sums * sums, axis=1, keepdims=True)
  safe = jnp.maximum(cnt, 1.0)
  per_class = (sumsq - normsq / safe) / safe
  present = cnt > 0.0
  n_unique = jnp.sum(present.astype(jnp.float32))
  out_ref[0, 0] = jnp.sum(jnp.where(present, per_class, 0.0)) / n_unique


def kernel(embeddings, labels):
  lab = labels.astype(jnp.int32)
  sums_p, sq_p, cnt_p = _sc_partials(embeddings, lab)
  loss = pl.pallas_call(
      _tc_combine_body,
      out_shape=jax.ShapeDtypeStruct((1, 1), jnp.float32),
      out_specs=pl.BlockSpec(memory_space=pltpu.SMEM),
  )(sums_p, sq_p, cnt_p)
  return loss[0, 0]


# speedup vs baseline: 1.2140x; 1.2140x over previous
import functools

import jax
import jax.numpy as jnp
from jax import lax
from jax.experimental import pallas as pl
from jax.experimental.pallas import tpu as pltpu
from jax.experimental.pallas import tpu_sc as plsc

N = 16384
D = 64
KPAD = 128
NC = 2
NS = 16
NW = NC * NS
CHUNK = N // NW
NB = CHUNK // 128
CW = 16
PB = 2048


def _pack_body(embt_ref, out_ref):
  xt = embt_ref[...].T
  out_ref[...] = jnp.concatenate([xt, xt * xt], axis=1)


def _sc_body(pk_hbm, lab_hbm, out_acc, out_cnt,
             lab_v, pk_v, ones_v, zb128, zb16, sh_acc, sh_cnt):
  c = lax.axis_index("c")
  s = lax.axis_index("s")
  wid = s * NC + c
  base = wid * CHUNK

  zv = jnp.zeros((16,), jnp.float32)
  ov = jnp.ones((16,), jnp.float32)

  rows = KPAD // NS
  for i in range(rows):
    for j in range(8):
      zb128[i, pl.ds(j * 16, 16)] = zv
    zb16[i, :] = zv
  pltpu.sync_copy(zb128, sh_acc.at[pl.ds(s * rows, rows)])
  pltpu.sync_copy(zb16, sh_cnt.at[pl.ds(s * rows, rows)])

  def ones_body(i, carry):
    ones_v[i, :] = ov
    return carry
  lax.fori_loop(0, 128, ones_body, 0)

  pltpu.sync_copy(pk_hbm.at[pl.ds(base, CHUNK)], pk_v)
  for b in range(NB):
    pltpu.sync_copy(lab_hbm.at[pl.ds(base + b * 128, 128)], lab_v.at[b])

  plsc.subcore_barrier()

  for b in range(NB):
    pltpu.sync_copy(pk_v.at[pl.ds(b * 128, 128)],
                    sh_acc.at[lab_v.at[b]], add=True)
    pltpu.sync_copy(ones_v, sh_cnt.at[lab_v.at[b]], add=True)

  plsc.subcore_barrier()

  @pl.when(s == 0)
  def _dump():
    pltpu.sync_copy(sh_acc, out_acc.at[c])
    pltpu.sync_copy(sh_cnt, out_cnt.at[c])


@functools.partial(
    pl.kernel,
    out_type=(
        jax.ShapeDtypeStruct((NC, KPAD, 2 * D), jnp.float32),
        jax.ShapeDtypeStruct((NC, KPAD, CW), jnp.float32),
    ),
    mesh=plsc.VectorSubcoreMesh(
        core_axis_name="c", subcore_axis_name="s",
        num_cores=NC, num_subcores=NS),
    compiler_params=pltpu.CompilerParams(use_tc_tiling_on_sc=False),
    scratch_types=[
        pltpu.VMEM((NB, 128), jnp.int32),
        pltpu.VMEM((CHUNK, 2 * D), jnp.float32),
        pltpu.VMEM((128, CW), jnp.float32),
        pltpu.VMEM((KPAD // NS, 2 * D), jnp.float32),
        pltpu.VMEM((KPAD // NS, CW), jnp.float32),
        pltpu.VMEM_SHARED((KPAD, 2 * D), jnp.float32),
        pltpu.VMEM_SHARED((KPAD, CW), jnp.float32),
    ],
)
def _sc_partials(pk_hbm, lab_hbm, out_acc, out_cnt, *scratch):
  _sc_body(pk_hbm, lab_hbm, out_acc, out_cnt, *scratch)


def _tc_combine_body(acc_ref, cnt_ref, out_ref):
  acc = acc_ref[0] + acc_ref[1]
  sums = acc[:, :D]
  sq = acc[:, D:]
  cnt = cnt_ref[0, :, 0:1] + cnt_ref[1, :, 0:1]
  sumsq = jnp.sum(sq, axis=1, keepdims=True)
  normsq = jnp.sum(sums * sums, axis=1, keepdims=True)
  safe = jnp.maximum(cnt, 1.0)
  per_class = (sumsq - normsq / safe) / safe
  present = cnt > 0.0
  n_unique = jnp.sum(present.astype(jnp.float32))
  out_ref[0, 0] = jnp.sum(jnp.where(present, per_class, 0.0)) / n_unique


def kernel(embeddings, labels):
  lab = labels.astype(jnp.int32)
  packed = pl.pallas_call(
      _pack_body,
      grid=(N // PB,),
      in_specs=[pl.BlockSpec((D, PB), lambda i: (0, i))],
      out_specs=pl.BlockSpec((PB, 2 * D), lambda i: (i, 0)),
      out_shape=jax.ShapeDtypeStruct((N, 2 * D), jnp.float32),
  )(embeddings.T)
  acc_p, cnt_p = _sc_partials(packed, lab)
  loss = pl.pallas_call(
      _tc_combine_body,
      out_shape=jax.ShapeDtypeStruct((1, 1), jnp.float32),
      out_specs=pl.BlockSpec(memory_space=pltpu.SMEM),
  )(acc_p, cnt_p)
  return loss[0, 0]


# speedup vs baseline: 1.3714x; 1.1296x over previous
import functools

import jax
import jax.numpy as jnp
from jax import lax
from jax.experimental import pallas as pl
from jax.experimental.pallas import tpu as pltpu
from jax.experimental.pallas import tpu_sc as plsc

N = 16384
D = 64
KPAD = 128
NC = 2
NS = 16
NW = NC * NS
CHUNK = N // NW
NB = CHUNK // 128
CW = 16
PB = 4096


def _pack_body(embt_ref, out_ref):
  x = embt_ref[...]
  row = lax.broadcasted_iota(jnp.int32, (D, D), 0)
  col = lax.broadcasted_iota(jnp.int32, (D, D), 1)
  eye = jnp.where(row == col, 1.0, 0.0).astype(jnp.float32)
  xt = jax.lax.dot_general(x, eye, (((0,), (0,)), ((), ())),
                           preferred_element_type=jnp.float32)
  out_ref[...] = jnp.concatenate([xt, xt * xt], axis=1)


def _sc_body(pk_hbm, lab_hbm, out_acc, out_cnt,
             lab_v, pk_v, ones_v, zb128, zb16, sem_in, sem_sc,
             sh_acc, sh_cnt):
  c = lax.axis_index("c")
  s = lax.axis_index("s")
  wid = s * NC + c
  base = wid * CHUNK

  zv = jnp.zeros((16,), jnp.float32)
  ov = jnp.ones((16,), jnp.float32)

  cp_pk = pltpu.async_copy(pk_hbm.at[pl.ds(base, CHUNK)], pk_v, sem_in)
  cp_lab = pltpu.async_copy(lab_hbm.at[pl.ds(wid * NB, NB)], lab_v, sem_in)

  rows = KPAD // NS
  for i in range(rows):
    for j in range(8):
      zb128[i, pl.ds(j * 16, 16)] = zv
    zb16[i, :] = zv
  pltpu.sync_copy(zb128, sh_acc.at[pl.ds(s * rows, rows)])
  pltpu.sync_copy(zb16, sh_cnt.at[pl.ds(s * rows, rows)])

  def ones_body(i, carry):
    ones_v[i, :] = ov
    return carry
  lax.fori_loop(0, 128, ones_body, 0)

  plsc.subcore_barrier()
  cp_pk.wait()
  cp_lab.wait()

  cps = []
  for b in range(NB):
    cps.append(pltpu.async_copy(pk_v.at[pl.ds(b * 128, 128)],
                                sh_acc.at[lab_v.at[b]], sem_sc, add=True))
    cps.append(pltpu.async_copy(ones_v, sh_cnt.at[lab_v.at[b]],
                                sem_sc, add=True))
  for cp in cps:
    cp.wait()

  plsc.subcore_barrier()

  @pl.when(s == 0)
  def _dump():
    pltpu.sync_copy(sh_acc, out_acc.at[c])
    pltpu.sync_copy(sh_cnt, out_cnt.at[c])


@functools.partial(
    pl.kernel,
    out_type=(
        jax.ShapeDtypeStruct((NC, KPAD, 2 * D), jnp.float32),
        jax.ShapeDtypeStruct((NC, KPAD, CW), jnp.float32),
    ),
    mesh=plsc.VectorSubcoreMesh(
        core_axis_name="c", subcore_axis_name="s",
        num_cores=NC, num_subcores=NS),
    compiler_params=pltpu.CompilerParams(use_tc_tiling_on_sc=False),
    scratch_types=[
        pltpu.VMEM((NB, 128), jnp.int32),
        pltpu.VMEM((CHUNK, 2 * D), jnp.float32),
        pltpu.VMEM((128, CW), jnp.float32),
        pltpu.VMEM((KPAD // NS, 2 * D), jnp.float32),
        pltpu.VMEM((KPAD // NS, CW), jnp.float32),
        pltpu.SemaphoreType.DMA,
        pltpu.SemaphoreType.DMA,
        pltpu.VMEM_SHARED((KPAD, 2 * D), jnp.float32),
        pltpu.VMEM_SHARED((KPAD, CW), jnp.float32),
    ],
)
def _sc_partials(pk_hbm, lab_hbm, out_acc, out_cnt, *scratch):
  _sc_body(pk_hbm, lab_hbm, out_acc, out_cnt, *scratch)


def _tc_combine_body(acc_ref, cnt_ref, out_ref):
  acc = acc_ref[0] + acc_ref[1]
  sums = acc[:, :D]
  sq = acc[:, D:]
  cnt = cnt_ref[0, :, 0:1] + cnt_ref[1, :, 0:1]
  sumsq = jnp.sum(sq, axis=1, keepdims=True)
  normsq = jnp.sum(sums * sums, axis=1, keepdims=True)
  safe = jnp.maximum(cnt, 1.0)
  per_class = (sumsq - normsq / safe) / safe
  present = cnt > 0.0
  n_unique = jnp.sum(present.astype(jnp.float32))
  out_ref[0, 0] = jnp.sum(jnp.where(present, per_class, 0.0)) / n_unique


def kernel(embeddings, labels):
  lab = labels.astype(jnp.int32).reshape(NW * NB, 128)
  packed = pl.pallas_call(
      _pack_body,
      grid=(N // PB,),
      in_specs=[pl.BlockSpec((D, PB), lambda i: (0, i))],
      out_specs=pl.BlockSpec((PB, 2 * D), lambda i: (i, 0)),
      out_shape=jax.ShapeDtypeStruct((N, 2 * D), jnp.float32),
  )(embeddings.T)
  acc_p, cnt_p = _sc_partials(packed, lab)
  loss = pl.pallas_call(
      _tc_combine_body,
      out_shape=jax.ShapeDtypeStruct((1, 1), jnp.float32),
      out_specs=pl.BlockSpec(memory_space=pltpu.SMEM),
  )(acc_p, cnt_p)
  return loss[0, 0]


# speedup vs baseline: 1.3728x; 1.0010x over previous
import functools

import jax
import jax.numpy as jnp
from jax import lax
from jax.experimental import pallas as pl
from jax.experimental.pallas import tpu as pltpu
from jax.experimental.pallas import tpu_sc as plsc

N = 16384
D = 64
KPAD = 128
NC = 2
NS = 16
NW = NC * NS
CHUNK = N // NW
NB = CHUNK // 128
CW = 16
PB = 8192


def _pack_body(embt_ref, out_ref):
  x = embt_ref[...]
  row = lax.broadcasted_iota(jnp.int32, (D, D), 0)
  col = lax.broadcasted_iota(jnp.int32, (D, D), 1)
  eye = jnp.where(row == col, 1.0, 0.0).astype(jnp.float32)
  xt = jax.lax.dot_general(x, eye, (((0,), (0,)), ((), ())),
                           preferred_element_type=jnp.float32)
  out_ref[:, :D] = xt
  out_ref[:, D:] = xt * xt


def _sc_body(pk_hbm, lab_hbm, out_acc, out_cnt,
             lab_v, pk_v, ones_v, zb128, zb16,
             sem_b0, sem_b1, sem_b2, sem_b3, sem_lab, sem_sc,
             sh_acc, sh_cnt):
  c = lax.axis_index("c")
  s = lax.axis_index("s")
  wid = s * NC + c
  base = wid * CHUNK
  sems = [sem_b0, sem_b1, sem_b2, sem_b3]

  zv = jnp.zeros((16,), jnp.float32)
  ov = jnp.ones((16,), jnp.float32)

  cps_in = [
      pltpu.async_copy(pk_hbm.at[pl.ds(base + b * 128, 128)],
                       pk_v.at[pl.ds(b * 128, 128)], sems[b])
      for b in range(NB)
  ]
  cp_lab = pltpu.async_copy(lab_hbm.at[pl.ds(wid * NB, NB)], lab_v, sem_lab)

  rows = KPAD // NS
  for i in range(rows):
    for j in range(8):
      zb128[i, pl.ds(j * 16, 16)] = zv
    zb16[i, :] = zv
  pltpu.sync_copy(zb128, sh_acc.at[pl.ds(s * rows, rows)])
  pltpu.sync_copy(zb16, sh_cnt.at[pl.ds(s * rows, rows)])

  def ones_body(i, carry):
    ones_v[i, :] = ov
    return carry
  lax.fori_loop(0, 128, ones_body, 0)

  plsc.subcore_barrier()
  cp_lab.wait()

  cps = []
  for b in range(NB):
    cps_in[b].wait()
    cps.append(pltpu.async_copy(pk_v.at[pl.ds(b * 128, 128)],
                                sh_acc.at[lab_v.at[b]], sem_sc, add=True))
    cps.append(pltpu.async_copy(ones_v, sh_cnt.at[lab_v.at[b]],
                                sem_sc, add=True))
  for cp in cps:
    cp.wait()

  plsc.subcore_barrier()

  @pl.when(s == 0)
  def _dump():
    pltpu.sync_copy(sh_acc, out_acc.at[c])
    pltpu.sync_copy(sh_cnt, out_cnt.at[c])


@functools.partial(
    pl.kernel,
    out_type=(
        jax.ShapeDtypeStruct((NC, KPAD, 2 * D), jnp.float32),
        jax.ShapeDtypeStruct((NC, KPAD, CW), jnp.float32),
    ),
    mesh=plsc.VectorSubcoreMesh(
        core_axis_name="c", subcore_axis_name="s",
        num_cores=NC, num_subcores=NS),
    compiler_params=pltpu.CompilerParams(use_tc_tiling_on_sc=False),
    scratch_types=[
        pltpu.VMEM((NB, 128), jnp.int32),
        pltpu.VMEM((CHUNK, 2 * D), jnp.float32),
        pltpu.VMEM((128, CW), jnp.float32),
        pltpu.VMEM((KPAD // NS, 2 * D), jnp.float32),
        pltpu.VMEM((KPAD // NS, CW), jnp.float32),
        pltpu.SemaphoreType.DMA,
        pltpu.SemaphoreType.DMA,
        pltpu.SemaphoreType.DMA,
        pltpu.SemaphoreType.DMA,
        pltpu.SemaphoreType.DMA,
        pltpu.SemaphoreType.DMA,
        pltpu.VMEM_SHARED((KPAD, 2 * D), jnp.float32),
        pltpu.VMEM_SHARED((KPAD, CW), jnp.float32),
    ],
)
def _sc_partials(pk_hbm, lab_hbm, out_acc, out_cnt, *scratch):
  _sc_body(pk_hbm, lab_hbm, out_acc, out_cnt, *scratch)


def _tc_combine_body(acc_ref, cnt_ref, out_ref):
  acc = acc_ref[0] + acc_ref[1]
  sums = acc[:, :D]
  sq = acc[:, D:]
  cnt = cnt_ref[0, :, 0:1] + cnt_ref[1, :, 0:1]
  sumsq = jnp.sum(sq, axis=1, keepdims=True)
  normsq = jnp.sum(sums * sums, axis=1, keepdims=True)
  safe = jnp.maximum(cnt, 1.0)
  per_class = (sumsq - normsq / safe) / safe
  present = cnt > 0.0
  n_unique = jnp.sum(present.astype(jnp.float32))
  out_ref[0, 0] = jnp.sum(jnp.where(present, per_class, 0.0)) / n_unique


def kernel(embeddings, labels):
  lab = labels.astype(jnp.int32).reshape(NW * NB, 128)
  packed = pl.pallas_call(
      _pack_body,
      grid=(N // PB,),
      in_specs=[pl.BlockSpec((D, PB), lambda i: (0, i))],
      out_specs=pl.BlockSpec((PB, 2 * D), lambda i: (i, 0)),
      out_shape=jax.ShapeDtypeStruct((N, 2 * D), jnp.float32),
  )(embeddings.T)
  acc_p, cnt_p = _sc_partials(packed, lab)
  loss = pl.pallas_call(
      _tc_combine_body,
      out_shape=jax.ShapeDtypeStruct((1, 1), jnp.float32),
      out_specs=pl.BlockSpec(memory_space=pltpu.SMEM),
  )(acc_p, cnt_p)
  return loss[0, 0]


# speedup vs baseline: 1.3869x; 1.0103x over previous
import functools

import jax
import jax.numpy as jnp
from jax import lax
from jax.experimental import pallas as pl
from jax.experimental.pallas import tpu as pltpu
from jax.experimental.pallas import tpu_sc as plsc

N = 16384
D = 64
KPAD = 128
NC = 2
NS = 16
NW = NC * NS
H = N // 2
PCHUNK = H // NW
NB = PCHUNK // 128
CW = 16
PB = 2048


def _pack_body(embt_a_ref, embt_b_ref, out_ref):
  row = lax.broadcasted_iota(jnp.int32, (D, D), 0)
  col = lax.broadcasted_iota(jnp.int32, (D, D), 1)
  eye = jnp.where(row == col, 1.0, 0.0).astype(jnp.float32)
  dims = (((0,), (0,)), ((), ()))
  xa = jax.lax.dot_general(embt_a_ref[...], eye, dims,
                           preferred_element_type=jnp.float32)
  xb = jax.lax.dot_general(embt_b_ref[...], eye, dims,
                           preferred_element_type=jnp.float32)
  out_ref[:, :D] = xa
  out_ref[:, D:] = xb


def _sc_body(pk_hbm, lab_hbm, out_acc, out_cnt,
             lab_v, pk_v, xa_v, xb_v, qa_v, qb_v, ones_v, zb64, zb16,
             sem_b0, sem_b1, sem_la, sem_lb, sem_sc,
             sh_sums, sh_sq, sh_cnt):
  c = lax.axis_index("c")
  s = lax.axis_index("s")
  wid = s * NC + c
  pbase = wid * PCHUNK
  zv = jnp.zeros((16,), jnp.float32)
  ov = jnp.ones((16,), jnp.float32)
  in_sems = [sem_b0, sem_b1]

  cps_in = [
      pltpu.async_copy(pk_hbm.at[pl.ds(pbase + b * 128, 128)],
                       pk_v.at[pl.ds(b * 128, 128)], in_sems[b])
      for b in range(NB)
  ]
  hrows = H // 128
  cp_la = pltpu.async_copy(lab_hbm.at[pl.ds(wid * NB, NB)],
                           lab_v.at[pl.ds(0, NB)], sem_la)
  cp_lb = pltpu.async_copy(lab_hbm.at[pl.ds(hrows + wid * NB, NB)],
                           lab_v.at[pl.ds(NB, NB)], sem_lb)

  rows = KPAD // NS
  for i in range(rows):
    for j in range(4):
      zb64[i, pl.ds(j * 16, 16)] = zv
    zb16[i, :] = zv
  pltpu.sync_copy(zb64, sh_sums.at[pl.ds(s * rows, rows)])
  pltpu.sync_copy(zb64, sh_sq.at[pl.ds(s * rows, rows)])
  pltpu.sync_copy(zb16, sh_cnt.at[pl.ds(s * rows, rows)])

  def ones_body(i, carry):
    ones_v[i, :] = ov
    return carry
  lax.fori_loop(0, 128, ones_body, 0)

  plsc.subcore_barrier()
  cp_la.wait()
  cp_lb.wait()

  cps = []
  for b in range(NB):
    cps_in[b].wait()

    @plsc.parallel_loop(0, 128, unroll=4)
    def row_body(r):
      for j in range(D // 16):
        va = pk_v[b * 128 + r, pl.ds(j * 16, 16)]
        vb = pk_v[b * 128 + r, pl.ds(D + j * 16, 16)]
        xa_v[b, r, pl.ds(j * 16, 16)] = va
        xb_v[b, r, pl.ds(j * 16, 16)] = vb
        qa_v[b, r, pl.ds(j * 16, 16)] = va * va
        qb_v[b, r, pl.ds(j * 16, 16)] = vb * vb

    for src_x, src_q, lrow in ((xa_v, qa_v, b), (xb_v, qb_v, NB + b)):
      idx = lab_v.at[lrow]
      cps.append(pltpu.async_copy(src_x.at[b], sh_sums.at[idx],
                                  sem_sc, add=True))
      cps.append(pltpu.async_copy(src_q.at[b], sh_sq.at[idx],
                                  sem_sc, add=True))
      cps.append(pltpu.async_copy(ones_v, sh_cnt.at[idx], sem_sc, add=True))
  for cp in cps:
    cp.wait()

  plsc.subcore_barrier()

  @pl.when(s == 0)
  def _dump():
    pltpu.sync_copy(sh_sums, out_acc.at[c, :, pl.ds(0, D)])
    pltpu.sync_copy(sh_sq, out_acc.at[c, :, pl.ds(D, D)])
    pltpu.sync_copy(sh_cnt, out_cnt.at[c])


@functools.partial(
    pl.kernel,
    out_type=(
        jax.ShapeDtypeStruct((NC, KPAD, 2 * D), jnp.float32),
        jax.ShapeDtypeStruct((NC, KPAD, CW), jnp.float32),
    ),
    mesh=plsc.VectorSubcoreMesh(
        core_axis_name="c", subcore_axis_name="s",
        num_cores=NC, num_subcores=NS),
    compiler_params=pltpu.CompilerParams(use_tc_tiling_on_sc=False),
    scratch_types=[
        pltpu.VMEM((2 * NB, 128), jnp.int32),
        pltpu.VMEM((PCHUNK, 2 * D), jnp.float32),
        pltpu.VMEM((NB, 128, D), jnp.float32),
        pltpu.VMEM((NB, 128, D), jnp.float32),
        pltpu.VMEM((NB, 128, D), jnp.float32),
        pltpu.VMEM((NB, 128, D), jnp.float32),
        pltpu.VMEM((128, CW), jnp.float32),
        pltpu.VMEM((KPAD // NS, D), jnp.float32),
        pltpu.VMEM((KPAD // NS, CW), jnp.float32),
        pltpu.SemaphoreType.DMA,
        pltpu.SemaphoreType.DMA,
        pltpu.SemaphoreType.DMA,
        pltpu.SemaphoreType.DMA,
        pltpu.SemaphoreType.DMA,
        pltpu.VMEM_SHARED((KPAD, D), jnp.float32),
        pltpu.VMEM_SHARED((KPAD, D), jnp.float32),
        pltpu.VMEM_SHARED((KPAD, CW), jnp.float32),
    ],
)
def _sc_partials(pk_hbm, lab_hbm, out_acc, out_cnt, *scratch):
  _sc_body(pk_hbm, lab_hbm, out_acc, out_cnt, *scratch)


def _tc_combine_body(acc_ref, cnt_ref, out_ref):
  acc = acc_ref[0] + acc_ref[1]
  sums = acc[:, :D]
  sq = acc[:, D:]
  cnt = cnt_ref[0, :, 0:1] + cnt_ref[1, :, 0:1]
  sumsq = jnp.sum(sq, axis=1, keepdims=True)
  normsq = jnp.sum(sums * sums, axis=1, keepdims=True)
  safe = jnp.maximum(cnt, 1.0)
  per_class = (sumsq - normsq / safe) / safe
  present = cnt > 0.0
  n_unique = jnp.sum(present.astype(jnp.float32))
  out_ref[0, 0] = jnp.sum(jnp.where(present, per_class, 0.0)) / n_unique


def kernel(embeddings, labels):
  lab = labels.astype(jnp.int32).reshape(N // 128, 128)
  packed = pl.pallas_call(
      _pack_body,
      grid=(H // PB,),
      in_specs=[
          pl.BlockSpec((D, PB), lambda i: (0, i)),
          pl.BlockSpec((D, PB), lambda i: (0, i + H // PB)),
      ],
      out_specs=pl.BlockSpec((PB, 2 * D), lambda i: (i, 0)),
      out_shape=jax.ShapeDtypeStruct((H, 2 * D), jnp.float32),
  )(embeddings.T, embeddings.T)
  acc_p, cnt_p = _sc_partials(packed, lab)
  loss = pl.pallas_call(
      _tc_combine_body,
      out_shape=jax.ShapeDtypeStruct((1, 1), jnp.float32),
      out_specs=pl.BlockSpec(memory_space=pltpu.SMEM),
  )(acc_p, cnt_p)
  return loss[0, 0]


# speedup vs baseline: 1.4022x; 1.0110x over previous
import functools

import jax
import jax.numpy as jnp
from jax import lax
from jax.experimental import pallas as pl
from jax.experimental.pallas import tpu as pltpu
from jax.experimental.pallas import tpu_sc as plsc

N = 16384
D = 64
KPAD = 128
NC = 2
NS = 16
NW = NC * NS
H = N // 2
PCHUNK = H // NW
NB = PCHUNK // 128
CW = 16
PB = 4096


def _pack_body(embt_a_ref, embt_b_ref, out_ref):
  row = lax.broadcasted_iota(jnp.int32, (D, D), 0)
  col = lax.broadcasted_iota(jnp.int32, (D, D), 1)
  eye = jnp.where(row == col, 1.0, 0.0).astype(jnp.float32)
  dims = (((0,), (0,)), ((), ()))
  xa = jax.lax.dot_general(embt_a_ref[...], eye, dims,
                           preferred_element_type=jnp.float32)
  xb = jax.lax.dot_general(embt_b_ref[...], eye, dims,
                           preferred_element_type=jnp.float32)
  out_ref[:, :D] = xa
  out_ref[:, D:] = xb


def _sc_body(pk_hbm, lab_hbm, out_acc, out_cnt,
             lab_v, pk_v, xa_v, xb_v, qa_v, qb_v, ones_v, zb64, zb16,
             sem_b0, sem_b1, sem_la, sem_lb, sem_sc,
             sh_sums, sh_sq, sh_cnt):
  c = lax.axis_index("c")
  s = lax.axis_index("s")
  wid = s * NC + c
  pbase = wid * PCHUNK
  zv = jnp.zeros((16,), jnp.float32)
  ov = jnp.ones((16,), jnp.float32)
  in_sems = [sem_b0, sem_b1]

  cps_in = [
      pltpu.async_copy(pk_hbm.at[pl.ds(pbase + b * 128, 128)],
                       pk_v.at[pl.ds(b * 128, 128)], in_sems[b])
      for b in range(NB)
  ]
  hrows = H // 128
  cp_la = pltpu.async_copy(lab_hbm.at[pl.ds(wid * NB, NB)],
                           lab_v.at[pl.ds(0, NB)], sem_la)
  cp_lb = pltpu.async_copy(lab_hbm.at[pl.ds(hrows + wid * NB, NB)],
                           lab_v.at[pl.ds(NB, NB)], sem_lb)

  rows = KPAD // NS
  for i in range(rows):
    for j in range(4):
      zb64[i, pl.ds(j * 16, 16)] = zv
    zb16[i, :] = zv
  pltpu.sync_copy(zb64, sh_sums.at[pl.ds(s * rows, rows)])
  pltpu.sync_copy(zb64, sh_sq.at[pl.ds(s * rows, rows)])
  pltpu.sync_copy(zb16, sh_cnt.at[pl.ds(s * rows, rows)])

  def ones_body(i, carry):
    ones_v[i, :] = ov
    return carry
  lax.fori_loop(0, 128, ones_body, 0)

  plsc.subcore_barrier()
  cp_la.wait()
  cp_lb.wait()

  cps = []
  for b in range(NB):
    cps_in[b].wait()

    @plsc.parallel_loop(0, 128, unroll=4)
    def row_body(r):
      for j in range(D // 16):
        va = pk_v[b * 128 + r, pl.ds(j * 16, 16)]
        vb = pk_v[b * 128 + r, pl.ds(D + j * 16, 16)]
        xa_v[b, r, pl.ds(j * 16, 16)] = va
        xb_v[b, r, pl.ds(j * 16, 16)] = vb
        qa_v[b, r, pl.ds(j * 16, 16)] = va * va
        qb_v[b, r, pl.ds(j * 16, 16)] = vb * vb

    for src_x, src_q, lrow in ((xa_v, qa_v, b), (xb_v, qb_v, NB + b)):
      idx = lab_v.at[lrow]
      cps.append(pltpu.async_copy(src_x.at[b], sh_sums.at[idx],
                                  sem_sc, add=True))
      cps.append(pltpu.async_copy(src_q.at[b], sh_sq.at[idx],
                                  sem_sc, add=True))
      cps.append(pltpu.async_copy(ones_v, sh_cnt.at[idx], sem_sc, add=True))
  for cp in cps:
    cp.wait()

  plsc.subcore_barrier()

  @pl.when(s == 0)
  def _dump():
    pltpu.sync_copy(sh_sums, out_acc.at[c, :, pl.ds(0, D)])
    pltpu.sync_copy(sh_sq, out_acc.at[c, :, pl.ds(D, D)])
    pltpu.sync_copy(sh_cnt, out_cnt.at[c])


@functools.partial(
    pl.kernel,
    out_type=(
        jax.ShapeDtypeStruct((NC, KPAD, 2 * D), jnp.float32),
        jax.ShapeDtypeStruct((NC, KPAD, CW), jnp.float32),
    ),
    mesh=plsc.VectorSubcoreMesh(
        core_axis_name="c", subcore_axis_name="s",
        num_cores=NC, num_subcores=NS),
    compiler_params=pltpu.CompilerParams(use_tc_tiling_on_sc=False),
    scratch_types=[
        pltpu.VMEM((2 * NB, 128), jnp.int32),
        pltpu.VMEM((PCHUNK, 2 * D), jnp.float32),
        pltpu.VMEM((NB, 128, D), jnp.float32),
        pltpu.VMEM((NB, 128, D), jnp.float32),
        pltpu.VMEM((NB, 128, D), jnp.float32),
        pltpu.VMEM((NB, 128, D), jnp.float32),
        pltpu.VMEM((128, CW), jnp.float32),
        pltpu.VMEM((KPAD // NS, D), jnp.float32),
        pltpu.VMEM((KPAD // NS, CW), jnp.float32),
        pltpu.SemaphoreType.DMA,
        pltpu.SemaphoreType.DMA,
        pltpu.SemaphoreType.DMA,
        pltpu.SemaphoreType.DMA,
        pltpu.SemaphoreType.DMA,
        pltpu.VMEM_SHARED((KPAD, D), jnp.float32),
        pltpu.VMEM_SHARED((KPAD, D), jnp.float32),
        pltpu.VMEM_SHARED((KPAD, CW), jnp.float32),
    ],
)
def _sc_partials(pk_hbm, lab_hbm, out_acc, out_cnt, *scratch):
  _sc_body(pk_hbm, lab_hbm, out_acc, out_cnt, *scratch)


def _tc_combine_body(acc_ref, cnt_ref, out_ref):
  acc = acc_ref[0] + acc_ref[1]
  sums = acc[:, :D]
  sq = acc[:, D:]
  cnt = cnt_ref[0, :, 0:1] + cnt_ref[1, :, 0:1]
  sumsq = jnp.sum(sq, axis=1, keepdims=True)
  normsq = jnp.sum(sums * sums, axis=1, keepdims=True)
  safe = jnp.maximum(cnt, 1.0)
  per_class = (sumsq - normsq / safe) / safe
  present = cnt > 0.0
  n_unique = jnp.sum(present.astype(jnp.float32))
  out_ref[0, 0] = jnp.sum(jnp.where(present, per_class, 0.0)) / n_unique


def kernel(embeddings, labels):
  lab = labels.astype(jnp.int32).reshape(N // 128, 128)
  packed = pl.pallas_call(
      _pack_body,
      grid=(H // PB,),
      in_specs=[
          pl.BlockSpec((D, PB), lambda i: (0, i)),
          pl.BlockSpec((D, PB), lambda i: (0, i + H // PB)),
      ],
      out_specs=pl.BlockSpec((PB, 2 * D), lambda i: (i, 0)),
      out_shape=jax.ShapeDtypeStruct((H, 2 * D), jnp.float32),
  )(embeddings.T, embeddings.T)
  acc_p, cnt_p = _sc_partials(packed, lab)
  loss = pl.pallas_call(
      _tc_combine_body,
      out_shape=jax.ShapeDtypeStruct((1, 1), jnp.float32),
      out_specs=pl.BlockSpec(memory_space=pltpu.SMEM),
  )(acc_p, cnt_p)
  return loss[0, 0]
